# Initial kernel scaffold; baseline (speedup 1.0000x reference)
import jax
import jax.numpy as jnp
from jax.experimental import pallas as pl


def kernel(x, edge_index, batch, pocket_features, W0, b0, g0, be0, W1, b1, g1, be1, W2, b2, g2, be2, Wp1, bp1, Wp2, bp2, Wbil, bbil, Wc1, bc1, Wc2, bc2):
    raise NotImplementedError("write your pallas kernel here")


# speedup vs baseline: 2.4829x; 2.4829x over previous
import jax
import jax.numpy as jnp
from jax.experimental import pallas as pl


def kernel(x, edge_index, batch, pocket_features, W0, b0, g0, be0, W1, b1, g1, be1, W2, b2, g2, be2, Wp1, bp1, Wp2, bp2, Wbil, bbil, Wc1, bc1, Wc2, bc2):
    src, dst = edge_index[0], edge_index[1]
    n = x.shape[0]
    B = 256

    deg = jax.ops.segment_sum(jnp.ones_like(dst, dtype=jnp.float32), dst, num_segments=n) + 1.0
    dinv = jax.lax.rsqrt(deg)

    def agg(h):
        hp = h * dinv[:, None]
        acc = jax.ops.segment_sum(hp[src], dst, num_segments=n)
        return (hp + acc) * dinv[:, None]

    def bn_relu(h, g, be, eps=1e-5):
        m = h.mean(axis=0)
        v = h.var(axis=0)
        return jax.nn.relu((h - m) / jnp.sqrt(v + eps) * g + be)

    for (W, b, g, be) in [(W0, b0, g0, be0), (W1, b1, g1, be1), (W2, b2, g2, be2)]:
        x = agg(x @ W) + b
        x = bn_relu(x, g, be)

    counts = jax.ops.segment_sum(jnp.ones((n,), x.dtype), batch, num_segments=B)
    ligand_emb = jax.ops.segment_sum(x, batch, num_segments=B) / jnp.maximum(counts, 1.0)[:, None]

    pocket_h = jax.nn.relu(pocket_features @ Wp1 + bp1) @ Wp2 + bp2
    M = jnp.einsum("oij,j->oi", Wbil, pocket_h)
    interaction_emb = ligand_emb @ M.T + bbil
    h = jax.nn.relu(interaction_emb @ Wc1 + bc1)
    logits = (h @ Wc2 + bc2).squeeze(-1)
    return logits


# speedup vs baseline: 11.0265x; 4.4410x over previous
import functools

import jax
import jax.numpy as jnp
from jax import lax
from jax.experimental import pallas as pl
from jax.experimental.pallas import tpu as pltpu
from jax.experimental.pallas import tpu_sc as plsc

_N = 10000
_E = 320000
_D = 128
_B = 256

_NCORE = 2
_NSUB = 16
_NW = _NCORE * _NSUB
_CH = 128
_NCHUNK = 79
_EPW = _CH * _NCHUNK
_NACC = 10112
_RPT = _NACC // _NSUB
_NB = 1000
_GRID = _N // _NB


def _sc_mesh():
    return plsc.VectorSubcoreMesh(core_axis_name="c", subcore_axis_name="s")


@functools.partial(
    pl.kernel,
    mesh=_sc_mesh(),
    out_type=jax.ShapeDtypeStruct((_NCORE, _NACC, _D), jnp.float32),
    scratch_types=[
        pltpu.VMEM((_NCHUNK, _CH), jnp.int32),
        pltpu.VMEM((_CH, _D), jnp.float32),
        pltpu.VMEM_SHARED((_NACC, _D), jnp.float32),
    ],
)
def _sc_degree(dst3, ones_h, zeros_h, out, dst_v, ones_v, acc_sh):
    cid = lax.axis_index("c")
    sid = lax.axis_index("s")
    w = cid * _NSUB + sid
    pltpu.sync_copy(zeros_h, acc_sh.at[pl.ds(sid * _RPT, _RPT)])
    pltpu.sync_copy(ones_h, ones_v)
    pltpu.sync_copy(dst3.at[w], dst_v)
    plsc.subcore_barrier()

    def body(j, carry):
        pltpu.sync_copy(ones_v, acc_sh.at[dst_v.at[j]], add=True)
        return carry

    lax.fori_loop(0, _NCHUNK, body, 0)
    plsc.subcore_barrier()
    pltpu.sync_copy(acc_sh.at[pl.ds(sid * _RPT, _RPT)],
                    out.at[cid, pl.ds(sid * _RPT, _RPT)])


@functools.partial(
    pl.kernel,
    mesh=_sc_mesh(),
    out_type=jax.ShapeDtypeStruct((_NCORE, _NACC, _D), jnp.float32),
    scratch_types=[
        pltpu.VMEM((_NCHUNK, _CH), jnp.int32),
        pltpu.VMEM((_NCHUNK, _CH), jnp.int32),
        pltpu.VMEM((_CH, _D), jnp.float32),
        pltpu.VMEM_SHARED((_NACC, _D), jnp.float32),
    ],
)
def _sc_aggregate(hp, src3, dst3, zeros_h, out, src_v, dst_v, rows_v, acc_sh):
    cid = lax.axis_index("c")
    sid = lax.axis_index("s")
    w = cid * _NSUB + sid
    pltpu.sync_copy(zeros_h, acc_sh.at[pl.ds(sid * _RPT, _RPT)])
    pltpu.sync_copy(src3.at[w], src_v)
    pltpu.sync_copy(dst3.at[w], dst_v)
    plsc.subcore_barrier()

    def body(j, carry):
        pltpu.sync_copy(hp.at[src_v.at[j]], rows_v)
        pltpu.sync_copy(rows_v, acc_sh.at[dst_v.at[j]], add=True)
        return carry

    lax.fori_loop(0, _NCHUNK, body, 0)
    plsc.subcore_barrier()
    pltpu.sync_copy(acc_sh.at[pl.ds(sid * _RPT, _RPT)],
                    out.at[cid, pl.ds(sid * _RPT, _RPT)])


def _mm0scale(x, W0, dd0, dd1):
    def body(x_ref, w_ref, d0_ref, d1_ref, hp_ref, dinv_ref):
        dinv = lax.rsqrt(1.0 + d0_ref[...] + d1_ref[...])
        dinv_ref[...] = dinv
        hp_ref[...] = jnp.dot(x_ref[...], w_ref[...],
                              preferred_element_type=jnp.float32) * dinv

    return pl.pallas_call(
        body,
        grid=(_GRID,),
        in_specs=[
            pl.BlockSpec((_NB, _D), lambda i: (i, 0)),
            pl.BlockSpec((_D, _D), lambda i: (0, 0)),
            pl.BlockSpec((_NB, 1), lambda i: (i, 0)),
            pl.BlockSpec((_NB, 1), lambda i: (i, 0)),
        ],
        out_specs=[
            pl.BlockSpec((_NB, _D), lambda i: (i, 0)),
            pl.BlockSpec((_NB, 1), lambda i: (i, 0)),
        ],
        out_shape=[
            jax.ShapeDtypeStruct((_N, _D), jnp.float32),
            jax.ShapeDtypeStruct((_N, 1), jnp.float32),
        ],
    )(x, W0, dd0, dd1)


def _combine(hp, a0, a1, dinv, b2d):
    def body(hp_ref, a0_ref, a1_ref, dinv_ref, b_ref, out_ref, s_ref, ss_ref):
        out = (hp_ref[...] + a0_ref[...] + a1_ref[...]) * dinv_ref[...] + b_ref[...]
        out_ref[...] = out

        @pl.when(pl.program_id(0) == 0)
        def _():
            s_ref[...] = jnp.zeros_like(s_ref)
            ss_ref[...] = jnp.zeros_like(ss_ref)

        s_ref[...] += jnp.sum(out, axis=0, keepdims=True)
        ss_ref[...] += jnp.sum(out * out, axis=0, keepdims=True)

    return pl.pallas_call(
        body,
        grid=(_GRID,),
        in_specs=[
            pl.BlockSpec((_NB, _D), lambda i: (i, 0)),
            pl.BlockSpec((_NB, _D), lambda i: (i, 0)),
            pl.BlockSpec((_NB, _D), lambda i: (i, 0)),
            pl.BlockSpec((_NB, 1), lambda i: (i, 0)),
            pl.BlockSpec((1, _D), lambda i: (0, 0)),
        ],
        out_specs=[
            pl.BlockSpec((_NB, _D), lambda i: (i, 0)),
            pl.BlockSpec((1, _D), lambda i: (0, 0)),
            pl.BlockSpec((1, _D), lambda i: (0, 0)),
        ],
        out_shape=[
            jax.ShapeDtypeStruct((_N, _D), jnp.float32),
            jax.ShapeDtypeStruct((1, _D), jnp.float32),
            jax.ShapeDtypeStruct((1, _D), jnp.float32),
        ],
    )(hp, a0, a1, dinv, b2d)


def _bn_relu_block(o, s, ss, g, be):
    m = s * (1.0 / _N)
    v = ss * (1.0 / _N) - m * m
    inv = lax.rsqrt(v + 1e-5)
    return jnp.maximum((o - m) * inv * g + be, 0.0)


def _bnmm(out_prev, s, ss, g2d, be2d, W, dinv):
    def body(o_ref, s_ref, ss_ref, g_ref, be_ref, w_ref, dinv_ref, hp_ref):
        xb = _bn_relu_block(o_ref[...], s_ref[...], ss_ref[...], g_ref[...], be_ref[...])
        hp_ref[...] = jnp.dot(xb, w_ref[...],
                              preferred_element_type=jnp.float32) * dinv_ref[...]

    return pl.pallas_call(
        body,
        grid=(_GRID,),
        in_specs=[
            pl.BlockSpec((_NB, _D), lambda i: (i, 0)),
            pl.BlockSpec((1, _D), lambda i: (0, 0)),
            pl.BlockSpec((1, _D), lambda i: (0, 0)),
            pl.BlockSpec((1, _D), lambda i: (0, 0)),
            pl.BlockSpec((1, _D), lambda i: (0, 0)),
            pl.BlockSpec((_D, _D), lambda i: (0, 0)),
            pl.BlockSpec((_NB, 1), lambda i: (i, 0)),
        ],
        out_specs=pl.BlockSpec((_NB, _D), lambda i: (i, 0)),
        out_shape=jax.ShapeDtypeStruct((_N, _D), jnp.float32),
    )(out_prev, s, ss, g2d, be2d, W, dinv)


def _pool(out2, s, ss, g2d, be2d, batch3):
    def body(o_ref, s_ref, ss_ref, g_ref, be_ref, bt_ref, pooled_ref, cnt_ref):
        xb = _bn_relu_block(o_ref[...], s_ref[...], ss_ref[...], g_ref[...], be_ref[...])
        bt = bt_ref[...].reshape(1, _NB)
        onehot = (lax.broadcasted_iota(jnp.int32, (_B, _NB), 0) == bt
                  ).astype(jnp.float32)

        @pl.when(pl.program_id(0) == 0)
        def _():
            pooled_ref[...] = jnp.zeros_like(pooled_ref)
            cnt_ref[...] = jnp.zeros_like(cnt_ref)

        pooled_ref[...] += jnp.dot(onehot, xb, preferred_element_type=jnp.float32)
        cnt_ref[...] += jnp.sum(onehot, axis=1, keepdims=True)

    return pl.pallas_call(
        body,
        grid=(_GRID,),
        in_specs=[
            pl.BlockSpec((_NB, _D), lambda i: (i, 0)),
            pl.BlockSpec((1, _D), lambda i: (0, 0)),
            pl.BlockSpec((1, _D), lambda i: (0, 0)),
            pl.BlockSpec((1, _D), lambda i: (0, 0)),
            pl.BlockSpec((1, _D), lambda i: (0, 0)),
            pl.BlockSpec((1, 1, _NB), lambda i: (i, 0, 0)),
        ],
        out_specs=[
            pl.BlockSpec((_B, _D), lambda i: (0, 0)),
            pl.BlockSpec((_B, 1), lambda i: (0, 0)),
        ],
        out_shape=[
            jax.ShapeDtypeStruct((_B, _D), jnp.float32),
            jax.ShapeDtypeStruct((_B, 1), jnp.float32),
        ],
    )(out2, s, ss, g2d, be2d, batch3)


def _tail1(pf2d, Wp1, bp1_2d, Wp2, bp2_2d, Wbil_io):
    def body(pf_ref, wp1_ref, b1_ref, wp2_ref, b2_ref, wb_ref, mflat_ref):
        ph = jnp.dot(
            jnp.maximum(jnp.dot(pf_ref[...], wp1_ref[...],
                                preferred_element_type=jnp.float32) + b1_ref[...], 0.0),
            wp2_ref[...], preferred_element_type=jnp.float32) + b2_ref[...]
        mflat_ref[...] = lax.dot_general(
            wb_ref[...], ph, (((1,), (1,)), ((), ())),
            preferred_element_type=jnp.float32)

    return pl.pallas_call(
        body,
        out_shape=jax.ShapeDtypeStruct((_D * 64, 1), jnp.float32),
    )(pf2d, Wp1, bp1_2d, Wp2, bp2_2d, Wbil_io)


def _tail2(pooled, cnt, M, bbil2d, Wc1, bc1_2d, Wc2, bc2_2d):
    def body(p_ref, c_ref, m_ref, bb_ref, wc1_ref, b1_ref, wc2_ref, b2_ref, out_ref):
        lig = p_ref[...] / jnp.maximum(c_ref[...], 1.0)
        inter = jnp.dot(lig, m_ref[...],
                        preferred_element_type=jnp.float32) + bb_ref[...]
        h = jnp.maximum(jnp.dot(inter, wc1_ref[...],
                                preferred_element_type=jnp.float32) + b1_ref[...], 0.0)
        out_ref[...] = jnp.dot(h, wc2_ref[...],
                               preferred_element_type=jnp.float32) + b2_ref[...]

    return pl.pallas_call(
        body,
        out_shape=jax.ShapeDtypeStruct((_B, 1), jnp.float32),
    )(pooled, cnt, M, bbil2d, Wc1, bc1_2d, Wc2, bc2_2d)


def kernel(x, edge_index, batch, pocket_features, W0, b0, g0, be0, W1, b1, g1, be1, W2, b2, g2, be2, Wp1, bp1, Wp2, bp2, Wbil, bbil, Wc1, bc1, Wc2, bc2):
    src, dst = edge_index[0], edge_index[1]

    pad = _NW * _EPW - _E
    srcp = jnp.concatenate([src, jnp.zeros((pad,), jnp.int32)])
    dstp = jnp.concatenate([dst, jnp.full((pad,), _NACC - 1, jnp.int32)])
    src3 = srcp.reshape(_NW, _NCHUNK, _CH)
    dst3 = dstp.reshape(_NW, _NCHUNK, _CH)

    onesD = jnp.ones((_CH, _D), jnp.float32)
    zerosD = jnp.zeros((_RPT, _D), jnp.float32)

    degacc = _sc_degree(dst3, onesD, zerosD)
    dd0 = degacc[0, :_N, 0:1]
    dd1 = degacc[1, :_N, 0:1]

    hp, dinv = _mm0scale(x, W0, dd0, dd1)

    outs_stats = []
    for (b, W_next) in [(b0, W1), (b1, W2), (b2, None)]:
        acc = _sc_aggregate(hp, src3, dst3, zerosD)
        out, s, ss = _combine(hp, acc[0, :_N], acc[1, :_N], dinv, b.reshape(1, _D))
        outs_stats.append((out, s, ss))
        if W_next is not None:
            g, be = (g0, be0) if W_next is W1 else (g1, be1)
            hp = _bnmm(out, s, ss, g.reshape(1, _D), be.reshape(1, _D), W_next, dinv)

    out2, s2, ss2 = outs_stats[-1]
    batch3 = batch.reshape(_GRID, 1, _NB)
    pooled, counts = _pool(out2, s2, ss2, g2.reshape(1, _D), be2.reshape(1, _D), batch3)

    Wbil_io = Wbil.transpose(1, 0, 2).reshape(_D * 64, 64)
    mflat = _tail1(pocket_features.reshape(1, -1), Wp1, bp1.reshape(1, -1),
                   Wp2, bp2.reshape(1, -1), Wbil_io)
    M = mflat.reshape(_D, 64)
    out = _tail2(pooled, counts, M, bbil.reshape(1, -1),
                 Wc1, bc1.reshape(1, -1), Wc2, bc2.reshape(1, -1))
    return out.reshape(_B)
